# baseline (speedup 1.0000x reference)
import functools

import jax
import jax.numpy as jnp
import numpy as np
from jax import lax
from jax.experimental import pallas as pl
from jax.experimental.pallas import tpu as pltpu
from jax.experimental.pallas import tpu_sc as plsc

_NC = 2
_NS = 16
_K = 128
_R = 2000

_HI = lax.Precision.HIGHEST


def _im2col_tables():
    g1 = np.full((16, 288), 288, np.int32)
    for c in range(32):
        for i in range(3):
            for j in range(3):
                for u in range(3):
                    for v in range(3):
                        i1, j1 = i + u - 1, j + v - 1
                        if 0 <= i1 < 3 and 0 <= j1 < 3:
                            g1[i1 * 3 + j1, c * 9 + i * 3 + j] = \
                                (c * 3 + u) * 3 + v
    g2 = np.full((288, 1024), 8192, np.int32)
    for c2 in range(64):
        for c1_ in range(32):
            for i2 in range(4):
                for j2 in range(4):
                    for u in range(2):
                        for v in range(2):
                            i1, j1 = i2 + u - 1, j2 + v - 1
                            if 0 <= i1 < 3 and 0 <= j1 < 3:
                                g2[c1_ * 9 + i1 * 3 + j1,
                                   c2 * 16 + i2 * 4 + j2] = \
                                    ((c2 * 32 + c1_) * 2 + u) * 2 + v
    pool = np.zeros((1024, 256), np.float32)
    for c in range(64):
        for i in range(4):
            for j in range(4):
                pool[c * 16 + i * 4 + j, c * 4 + (i // 2) * 2 + j // 2] = 0.25
    return g1, g2, pool


_G1, _G2, _POOL = _im2col_tables()


def _fold_weights(conv1_w, conv1_b, conv2_w, conv2_b, fcfe_w, fcfe_b, gcn1_w):
    w1p = jnp.append(conv1_w.reshape(-1), 0.0)[_G1]
    b1 = jnp.repeat(conv1_b, 9)
    w2 = jnp.append(conv2_w.reshape(-1), 0.0)[_G2]
    b2 = jnp.repeat(conv2_b, 16)
    wf = jnp.asarray(_POOL) @ (fcfe_w @ gcn1_w)
    bf = fcfe_b @ gcn1_w
    return w1p, b1, w2, b2, wf, bf


def _acc_geom(n):
    opt = -(-(-(-n // _NS)) // 8) * 8
    n_out = _NS * opt
    zpt = -(-max(n_out, n + 1) // _NS // 8) * 8
    n_acc = _NS * zpt
    return opt, n_out, zpt, n_acc


def _sc_s1(scat_idx, ones_rows, zrows, n, e_pad):
    opt, n_out, zpt, n_acc = _acc_geom(n)
    chunks = e_pad // (_NS * _K)
    mesh = plsc.VectorSubcoreMesh(core_axis_name="c", subcore_axis_name="s",
                                  num_cores=_NC, num_subcores=_NS)

    @functools.partial(
        pl.kernel, mesh=mesh,
        out_type=[jax.ShapeDtypeStruct((n_out, 16), jnp.float32),
                  jax.ShapeDtypeStruct((n_out, 16), jnp.float32)],
        scratch_types=[
            pltpu.VMEM_SHARED((n_acc, 16), jnp.float32),
            pltpu.VMEM((_K,), jnp.int32),
            pltpu.VMEM((_K, 16), jnp.float32),
        ],
        compiler_params=pltpu.CompilerParams(use_tc_tiling_on_sc=False))
    def s1(scat_idx_hbm, ones_hbm, zrows_hbm, deg_hbm, cnt_hbm,
           acc, idx_v, ones_v):
        cid = lax.axis_index("c")
        sid = lax.axis_index("s")
        pltpu.sync_copy(zrows_hbm, acc.at[pl.ds(sid * zpt, zpt)])
        pltpu.sync_copy(ones_hbm, ones_v)
        plsc.subcore_barrier()
        for j in range(chunks):
            off = cid * e_pad + sid * (chunks * _K) + j * _K
            pltpu.sync_copy(scat_idx_hbm.at[pl.ds(off, _K)], idx_v)
            pltpu.sync_copy(ones_v, acc.at[idx_v], add=True)
        plsc.subcore_barrier()

        @pl.when(cid == 0)
        def _():
            pltpu.sync_copy(acc.at[pl.ds(sid * opt, opt)],
                            deg_hbm.at[pl.ds(sid * opt, opt)])

        @pl.when(cid == 1)
        def _():
            pltpu.sync_copy(acc.at[pl.ds(sid * opt, opt)],
                            cnt_hbm.at[pl.ds(sid * opt, opt)])

    return s1(scat_idx, ones_rows, zrows)


def _sc_agg(ya, yb, src_g, dst_g, dst_s, nodeinfo, zrows, n, e_pad, with_info):
    opt, n_out, zpt, n_acc = _acc_geom(n)
    chunks = e_pad // (_NS * _K)
    mesh = plsc.VectorSubcoreMesh(core_axis_name="c", subcore_axis_name="s",
                                  num_cores=_NC, num_subcores=_NS)
    out_type = [jax.ShapeDtypeStruct((n_out, 32), jnp.float32),
                jax.ShapeDtypeStruct((n_out, 32), jnp.float32)]
    if with_info:
        out_type += [jax.ShapeDtypeStruct((e_pad, 16), jnp.float32),
                     jax.ShapeDtypeStruct((e_pad, 16), jnp.float32)]

    @functools.partial(
        pl.kernel, mesh=mesh,
        out_type=out_type,
        scratch_types=[
            pltpu.VMEM_SHARED((n_acc, 32), jnp.float32),
            pltpu.VMEM((_K,), jnp.int32),
            pltpu.VMEM((_K,), jnp.int32),
            pltpu.VMEM((_K,), jnp.int32),
            pltpu.VMEM((_K, 32), jnp.float32),
            pltpu.VMEM((_K, 16), jnp.float32),
            pltpu.SemaphoreType.DMA,
        ],
        compiler_params=pltpu.CompilerParams(use_tc_tiling_on_sc=False))
    def agg_kernel(ya_hbm, yb_hbm, src_g_hbm, dst_g_hbm, dst_s_hbm,
                   info_hbm, zrows_hbm, *out_and_scratch):
        if with_info:
            agga_hbm, aggb_hbm, isrc_hbm, idst_hbm = out_and_scratch[:4]
            rest = out_and_scratch[4:]
        else:
            agga_hbm, aggb_hbm = out_and_scratch[:2]
            rest = out_and_scratch[2:]
        acc, sidx_v, didx_v, gidx_v, rows_v, info_v, sem = rest
        cid = lax.axis_index("c")
        sid = lax.axis_index("s")
        pltpu.sync_copy(zrows_hbm, acc.at[pl.ds(sid * zpt, zpt)])
        plsc.subcore_barrier()
        for j in range(chunks):
            off = sid * (chunks * _K) + j * _K
            pltpu.sync_copy(src_g_hbm.at[pl.ds(off, _K)], sidx_v)
            pltpu.sync_copy(dst_s_hbm.at[pl.ds(off, _K)], didx_v)

            @pl.when(cid == 0)
            def _():
                pltpu.async_copy(ya_hbm.at[sidx_v], rows_v, sem).wait()

            @pl.when(cid == 1)
            def _():
                pltpu.async_copy(yb_hbm.at[sidx_v], rows_v, sem).wait()

            pltpu.sync_copy(rows_v, acc.at[didx_v], add=True)
            if with_info:
                @pl.when(cid == 0)
                def _():
                    pltpu.async_copy(info_hbm.at[sidx_v], info_v, sem).wait()
                    pltpu.sync_copy(info_v, isrc_hbm.at[pl.ds(off, _K)])

                @pl.when(cid == 1)
                def _():
                    pltpu.sync_copy(dst_g_hbm.at[pl.ds(off, _K)], gidx_v)
                    pltpu.async_copy(info_hbm.at[gidx_v], info_v, sem).wait()
                    pltpu.sync_copy(info_v, idst_hbm.at[pl.ds(off, _K)])
        plsc.subcore_barrier()

        @pl.when(cid == 0)
        def _():
            pltpu.sync_copy(acc.at[pl.ds(sid * opt, opt)],
                            agga_hbm.at[pl.ds(sid * opt, opt)])

        @pl.when(cid == 1)
        def _():
            pltpu.sync_copy(acc.at[pl.ds(sid * opt, opt)],
                            aggb_hbm.at[pl.ds(sid * opt, opt)])

    return agg_kernel(ya, yb, src_g, dst_g, dst_s, nodeinfo, zrows)


def _tc_main(x16, deg_col, theta2d, r2d, w1p, b1, w2, b2, wf, bf, n, n2d):
    nb = n // _R
    rb = n2d // nb

    def body(x_ref, deg_ref, th_ref, r_ref, w1_ref, b1_ref, w2_ref, b2_ref,
             wf_ref, bf_ref, ya_ref, yb_ref, dinv_ref, px_ref, py_ref):
        h1 = jnp.maximum(
            jnp.dot(x_ref[...], w1_ref[...],
                    preferred_element_type=jnp.float32) + b1_ref[...], 0.0)
        h2 = jnp.maximum(
            jnp.dot(h1.astype(jnp.bfloat16), w2_ref[...],
                    preferred_element_type=jnp.float32) + b2_ref[...], 0.0)
        xw = jnp.dot(h2.astype(jnp.bfloat16), wf_ref[...],
                     preferred_element_type=jnp.float32) + bf_ref[...]
        dinv = lax.rsqrt(deg_ref[...] + 1.0)
        dinv_ref[...] = dinv
        y1 = dinv * xw
        ya_ref[...] = y1[:, :32]
        yb_ref[...] = y1[:, 32:]
        px_ref[...] = r_ref[...] * jnp.cos(th_ref[...])
        py_ref[...] = r_ref[...] * jnp.sin(th_ref[...])

    return pl.pallas_call(
        body,
        grid=(nb,),
        in_specs=[
            pl.BlockSpec((_R, 16), lambda i: (i, 0)),
            pl.BlockSpec((_R, 1), lambda i: (i, 0)),
            pl.BlockSpec((rb, 128), lambda i: (i, 0)),
            pl.BlockSpec((rb, 128), lambda i: (i, 0)),
            pl.BlockSpec((16, 288), lambda i: (0, 0)),
            pl.BlockSpec((1, 288), lambda i: (0, 0)),
            pl.BlockSpec((288, 1024), lambda i: (0, 0)),
            pl.BlockSpec((1, 1024), lambda i: (0, 0)),
            pl.BlockSpec((1024, 64), lambda i: (0, 0)),
            pl.BlockSpec((1, 64), lambda i: (0, 0)),
        ],
        out_specs=[
            pl.BlockSpec((_R, 32), lambda i: (i, 0)),
            pl.BlockSpec((_R, 32), lambda i: (i, 0)),
            pl.BlockSpec((_R, 1), lambda i: (i, 0)),
            pl.BlockSpec((rb, 128), lambda i: (i, 0)),
            pl.BlockSpec((rb, 128), lambda i: (i, 0)),
        ],
        out_shape=[
            jax.ShapeDtypeStruct((n, 32), jnp.float32),
            jax.ShapeDtypeStruct((n, 32), jnp.float32),
            jax.ShapeDtypeStruct((n, 1), jnp.float32),
            jax.ShapeDtypeStruct((n2d, 128), jnp.float32),
            jax.ShapeDtypeStruct((n2d, 128), jnp.float32),
        ],
    )(x16, deg_col, theta2d, r2d, w1p, b1.reshape(1, 288),
      w2.astype(jnp.bfloat16), b2.reshape(1, 1024), wf.astype(jnp.bfloat16),
      bf.reshape(1, 64))


def _tc_edge(fs, cs, pxs, pys, fd, pxd, pyd, n2d):
    nb = 25
    rb = n2d // nb

    def body(fs_ref, cs_ref, pxs_ref, pys_ref, fd_ref, pxd_ref, pyd_ref,
             a1_ref, a2_ref):
        df = jnp.abs(fs_ref[...] - fd_ref[...])
        a1_ref[...] = jnp.where(df == 1.0,
                                1.0 / jnp.maximum(cs_ref[...], 1.0), 0.0)
        d2 = (pxd_ref[...] - pxs_ref[...]) ** 2 + \
             (pyd_ref[...] - pys_ref[...]) ** 2
        disp = jnp.sqrt(jnp.maximum(d2, 1e-12))
        vel = disp / jnp.where(df == 2.0, df, 1.0)
        a2_ref[...] = jnp.where(df == 2.0, jnp.exp(-vel / 8.5), 0.0)

    spec = pl.BlockSpec((rb, 128), lambda i: (i, 0))
    return pl.pallas_call(
        body,
        grid=(nb,),
        in_specs=[spec] * 7,
        out_specs=[spec] * 2,
        out_shape=[jax.ShapeDtypeStruct((n2d, 128), jnp.float32)] * 2,
    )(fs, cs, pxs, pys, fd, pxd, pyd)


def _tc_mid(agg1a, agg1b, ya, yb, dinv_col, a1_col, gcn1_b, gcn2_w, n):
    nb = n // _R

    def body(aggA_ref, aggB_ref, ya_ref, yb_ref, dinv_ref, a1_ref,
             b1_ref, w2_ref, y2a_ref, y2b_ref):
        dinv = dinv_ref[...]
        agg = jnp.concatenate([aggA_ref[...], aggB_ref[...]], axis=1)
        y1 = jnp.concatenate([ya_ref[...], yb_ref[...]], axis=1)
        g1 = dinv * (agg + y1) + b1_ref[...]
        x1 = jnp.maximum(g1 * a1_ref[...], 0.0)
        y2 = dinv * jnp.dot(x1, w2_ref[...], precision=_HI)
        y2a_ref[...] = y2[:, :32]
        y2b_ref[...] = y2[:, 32:]

    return pl.pallas_call(
        body,
        grid=(nb,),
        in_specs=[
            pl.BlockSpec((_R, 32), lambda i: (i, 0)),
            pl.BlockSpec((_R, 32), lambda i: (i, 0)),
            pl.BlockSpec((_R, 32), lambda i: (i, 0)),
            pl.BlockSpec((_R, 32), lambda i: (i, 0)),
            pl.BlockSpec((_R, 1), lambda i: (i, 0)),
            pl.BlockSpec((_R, 1), lambda i: (i, 0)),
            pl.BlockSpec((1, 64), lambda i: (0, 0)),
            pl.BlockSpec((64, 64), lambda i: (0, 0)),
        ],
        out_specs=[
            pl.BlockSpec((_R, 32), lambda i: (i, 0)),
            pl.BlockSpec((_R, 32), lambda i: (i, 0)),
        ],
        out_shape=[
            jax.ShapeDtypeStruct((n, 32), jnp.float32),
            jax.ShapeDtypeStruct((n, 32), jnp.float32),
        ],
    )(agg1a, agg1b, ya, yb, dinv_col, a1_col,
      gcn1_b.reshape(1, 64), gcn2_w)


def _tc_head(agg2a, agg2b, y2a, y2b, dinv_col, a2_col, gcn2_b, fc_w, fc_b,
             out_w, out_b, n):
    nb = n // _R

    def body(aggA_ref, aggB_ref, ya_ref, yb_ref, dinv_ref, a2_ref, b2_ref,
             fw_ref, fb_ref, ow_ref, ob_ref, o_ref):
        dinv = dinv_ref[...]
        agg = jnp.concatenate([aggA_ref[...], aggB_ref[...]], axis=1)
        y2 = jnp.concatenate([ya_ref[...], yb_ref[...]], axis=1)
        g2 = dinv * (agg + y2) + b2_ref[...]
        x2 = jnp.maximum(g2 * a2_ref[...], 0.0)
        h = jnp.maximum(jnp.dot(x2, fw_ref[...], precision=_HI) + fb_ref[...],
                        0.0)
        logit = jnp.dot(h, ow_ref[...], precision=_HI) + ob_ref[...]
        o_ref[...] = 1.0 / (1.0 + jnp.exp(-logit))

    return pl.pallas_call(
        body,
        grid=(nb,),
        in_specs=[
            pl.BlockSpec((_R, 32), lambda i: (i, 0)),
            pl.BlockSpec((_R, 32), lambda i: (i, 0)),
            pl.BlockSpec((_R, 32), lambda i: (i, 0)),
            pl.BlockSpec((_R, 32), lambda i: (i, 0)),
            pl.BlockSpec((_R, 1), lambda i: (i, 0)),
            pl.BlockSpec((_R, 1), lambda i: (i, 0)),
            pl.BlockSpec((1, 64), lambda i: (0, 0)),
            pl.BlockSpec((64, 32), lambda i: (0, 0)),
            pl.BlockSpec((1, 32), lambda i: (0, 0)),
            pl.BlockSpec((32, 1), lambda i: (0, 0)),
            pl.BlockSpec((1, 1), lambda i: (0, 0)),
        ],
        out_specs=pl.BlockSpec((_R, 1), lambda i: (i, 0)),
        out_shape=jax.ShapeDtypeStruct((n, 1), jnp.float32),
    )(agg2a, agg2b, y2a, y2b, dinv_col, a2_col, gcn2_b.reshape(1, 64), fc_w,
      fc_b.reshape(1, 32), out_w, out_b.reshape(1, 1))


def kernel(x, edge_index, aux_features, conv1_w, conv1_b, conv2_w, conv2_b,
           fcfe_w, fcfe_b, gcn1_w, gcn1_b, gcn2_w, gcn2_b,
           fc_w, fc_b, out_w, out_b):
    n = x.shape[0]
    e = edge_index.shape[1]
    e_pad = -(-e // (_NS * _K)) * (_NS * _K)

    x16 = jnp.zeros((n, 16), jnp.float32).at[:, :9].set(x.reshape(n, 9))
    w1p, b1, w2, b2, wf, bf = _fold_weights(
        conv1_w, conv1_b, conv2_w, conv2_b, fcfe_w, fcfe_b, gcn1_w)
    src = edge_index[0]
    dst = edge_index[1]
    pad = e_pad - e
    pad_g = jnp.zeros((pad,), jnp.int32)
    pad_s = jnp.full((pad,), n, jnp.int32)
    src_g = jnp.concatenate([src, pad_g])
    dst_g = jnp.concatenate([dst, pad_g])
    src_s = jnp.concatenate([src, pad_s])
    dst_s = jnp.concatenate([dst, pad_s])
    scat_idx = jnp.concatenate([dst_s, src_s])
    ones_rows = jnp.zeros((_K, 16), jnp.float32).at[:, 0].set(1.0)
    zpt = _acc_geom(n)[2]
    zrows16 = jnp.zeros((zpt, 16), jnp.float32)
    zrows32 = jnp.zeros((zpt, 32), jnp.float32)

    n2d = e_pad // 128
    npad = e_pad - n

    def to2d(col, fill=0.0):
        return jnp.concatenate(
            [col, jnp.full((npad,), fill, jnp.float32)]).reshape(n2d, 128)

    theta2d = to2d(aux_features[:, 0])
    r2d = to2d(aux_features[:, 1])

    deg16, cnt16 = _sc_s1(scat_idx, ones_rows, zrows16, n, e_pad)
    deg_col = deg16[:n, 0:1]
    ya, yb, dinv_col, px2d, py2d = _tc_main(
        x16, deg_col, theta2d, r2d, w1p, b1, w2, b2, wf, bf, n, n2d)
    px_col = px2d.reshape(e_pad, 1)
    py_col = py2d.reshape(e_pad, 1)
    nodeinfo = jnp.concatenate(
        [aux_features[:, 2:3], cnt16[:n, 0:1], px_col[:n], py_col[:n],
         jnp.zeros((n, 12), jnp.float32)], axis=1)
    agg1a, agg1b, isrc, idst = _sc_agg(ya, yb, src_g, dst_g, dst_s,
                                       nodeinfo, zrows32, n, e_pad,
                                       with_info=True)
    a1_2d, a2_2d = _tc_edge(
        isrc[:, 0].reshape(n2d, 128), isrc[:, 1].reshape(n2d, 128),
        isrc[:, 2].reshape(n2d, 128), isrc[:, 3].reshape(n2d, 128),
        idst[:, 0].reshape(n2d, 128), idst[:, 2].reshape(n2d, 128),
        idst[:, 3].reshape(n2d, 128), n2d)
    a1_col = a1_2d.reshape(e_pad, 1)
    a2_col = a2_2d.reshape(e_pad, 1)
    y2a, y2b = _tc_mid(agg1a, agg1b, ya, yb, dinv_col, a1_col[:n],
                       gcn1_b, gcn2_w, n)
    agg2a, agg2b = _sc_agg(y2a, y2b, src_g, dst_g, dst_s, nodeinfo,
                           zrows32, n, e_pad, with_info=False)
    out = _tc_head(agg2a, agg2b, y2a, y2b, dinv_col, a2_col[:n], gcn2_b,
                   fc_w, fc_b, out_w, out_b, n)
    return out


# speedup vs baseline: 3.2387x; 3.2387x over previous
import functools

import jax
import jax.numpy as jnp
import numpy as np
from jax import lax
from jax.experimental import pallas as pl
from jax.experimental.pallas import tpu as pltpu
from jax.experimental.pallas import tpu_sc as plsc

_NC = 2
_NS = 16
_K = 128
_R = 2000

_HI = lax.Precision.HIGHEST


def _im2col_tables():
    g1 = np.full((16, 288), 288, np.int32)
    for c in range(32):
        for i in range(3):
            for j in range(3):
                for u in range(3):
                    for v in range(3):
                        i1, j1 = i + u - 1, j + v - 1
                        if 0 <= i1 < 3 and 0 <= j1 < 3:
                            g1[i1 * 3 + j1, c * 9 + i * 3 + j] = \
                                (c * 3 + u) * 3 + v
    g2 = np.full((288, 1024), 8192, np.int32)
    for c2 in range(64):
        for c1_ in range(32):
            for i2 in range(4):
                for j2 in range(4):
                    for u in range(2):
                        for v in range(2):
                            i1, j1 = i2 + u - 1, j2 + v - 1
                            if 0 <= i1 < 3 and 0 <= j1 < 3:
                                g2[c1_ * 9 + i1 * 3 + j1,
                                   c2 * 16 + i2 * 4 + j2] = \
                                    ((c2 * 32 + c1_) * 2 + u) * 2 + v
    pool = np.zeros((1024, 256), np.float32)
    for c in range(64):
        for i in range(4):
            for j in range(4):
                pool[c * 16 + i * 4 + j, c * 4 + (i // 2) * 2 + j // 2] = 0.25
    return g1, g2, pool


_G1, _G2, _POOL = _im2col_tables()


def _conv_nb(x, w, pad):
    return lax.conv_general_dilated(
        x, w, (1, 1), [(pad, pad), (pad, pad)],
        dimension_numbers=('NCHW', 'OIHW', 'NCHW'))


def _fold_weights(conv1_w, conv1_b, conv2_w, conv2_b, fcfe_w, fcfe_b, gcn1_w):
    eye9 = jnp.eye(9, dtype=jnp.float32).reshape(9, 1, 3, 3)
    w1 = _conv_nb(eye9, conv1_w, 1).reshape(9, 288)
    w1p = jnp.zeros((16, 288), jnp.float32).at[:9].set(w1)
    b1 = jnp.repeat(conv1_b, 9)
    eye288 = jnp.eye(288, dtype=jnp.float32).reshape(288, 32, 3, 3)
    w2 = _conv_nb(eye288, conv2_w, 1).reshape(288, 1024)
    b2 = jnp.repeat(conv2_b, 16)
    wf = jnp.asarray(_POOL) @ (fcfe_w @ gcn1_w)
    bf = fcfe_b @ gcn1_w
    return w1p, b1, w2, b2, wf, bf


def _acc_geom(n):
    opt = -(-(-(-n // _NS)) // 8) * 8
    n_out = _NS * opt
    zpt = -(-max(n_out, n + 1) // _NS // 8) * 8
    n_acc = _NS * zpt
    return opt, n_out, zpt, n_acc


def _sc_s1(scat_idx, ones_rows, zrows, n, e_pad):
    opt, n_out, zpt, n_acc = _acc_geom(n)
    chunks = e_pad // (_NS * _K)
    mesh = plsc.VectorSubcoreMesh(core_axis_name="c", subcore_axis_name="s",
                                  num_cores=_NC, num_subcores=_NS)

    @functools.partial(
        pl.kernel, mesh=mesh,
        out_type=[jax.ShapeDtypeStruct((n_out, 16), jnp.float32),
                  jax.ShapeDtypeStruct((n_out, 16), jnp.float32)],
        scratch_types=[
            pltpu.VMEM_SHARED((n_acc, 16), jnp.float32),
            pltpu.VMEM((_K,), jnp.int32),
            pltpu.VMEM((_K, 16), jnp.float32),
        ],
        compiler_params=pltpu.CompilerParams(use_tc_tiling_on_sc=False))
    def s1(scat_idx_hbm, ones_hbm, zrows_hbm, deg_hbm, cnt_hbm,
           acc, idx_v, ones_v):
        cid = lax.axis_index("c")
        sid = lax.axis_index("s")
        pltpu.sync_copy(zrows_hbm, acc.at[pl.ds(sid * zpt, zpt)])
        pltpu.sync_copy(ones_hbm, ones_v)
        plsc.subcore_barrier()
        for j in range(chunks):
            off = cid * e_pad + sid * (chunks * _K) + j * _K
            pltpu.sync_copy(scat_idx_hbm.at[pl.ds(off, _K)], idx_v)
            pltpu.sync_copy(ones_v, acc.at[idx_v], add=True)
        plsc.subcore_barrier()

        @pl.when(cid == 0)
        def _():
            pltpu.sync_copy(acc.at[pl.ds(sid * opt, opt)],
                            deg_hbm.at[pl.ds(sid * opt, opt)])

        @pl.when(cid == 1)
        def _():
            pltpu.sync_copy(acc.at[pl.ds(sid * opt, opt)],
                            cnt_hbm.at[pl.ds(sid * opt, opt)])

    return s1(scat_idx, ones_rows, zrows)


def _sc_agg(ya, yb, src_g, dst_g, dst_s, nodeinfo, zrows, n, e_pad, with_info):
    opt, n_out, zpt, n_acc = _acc_geom(n)
    chunks = e_pad // (_NS * _K)
    mesh = plsc.VectorSubcoreMesh(core_axis_name="c", subcore_axis_name="s",
                                  num_cores=_NC, num_subcores=_NS)
    out_type = [jax.ShapeDtypeStruct((n_out, 32), jnp.float32),
                jax.ShapeDtypeStruct((n_out, 32), jnp.float32)]
    if with_info:
        out_type += [jax.ShapeDtypeStruct((e_pad, 16), jnp.float32),
                     jax.ShapeDtypeStruct((e_pad, 16), jnp.float32)]

    @functools.partial(
        pl.kernel, mesh=mesh,
        out_type=out_type,
        scratch_types=[
            pltpu.VMEM_SHARED((n_acc, 32), jnp.float32),
            pltpu.VMEM((_K,), jnp.int32),
            pltpu.VMEM((_K,), jnp.int32),
            pltpu.VMEM((_K,), jnp.int32),
            pltpu.VMEM((_K, 32), jnp.float32),
            pltpu.VMEM((_K, 16), jnp.float32),
            pltpu.SemaphoreType.DMA,
        ],
        compiler_params=pltpu.CompilerParams(use_tc_tiling_on_sc=False))
    def agg_kernel(ya_hbm, yb_hbm, src_g_hbm, dst_g_hbm, dst_s_hbm,
                   info_hbm, zrows_hbm, *out_and_scratch):
        if with_info:
            agga_hbm, aggb_hbm, isrc_hbm, idst_hbm = out_and_scratch[:4]
            rest = out_and_scratch[4:]
        else:
            agga_hbm, aggb_hbm = out_and_scratch[:2]
            rest = out_and_scratch[2:]
        acc, sidx_v, didx_v, gidx_v, rows_v, info_v, sem = rest
        cid = lax.axis_index("c")
        sid = lax.axis_index("s")
        pltpu.sync_copy(zrows_hbm, acc.at[pl.ds(sid * zpt, zpt)])
        plsc.subcore_barrier()
        for j in range(chunks):
            off = sid * (chunks * _K) + j * _K
            pltpu.sync_copy(src_g_hbm.at[pl.ds(off, _K)], sidx_v)
            pltpu.sync_copy(dst_s_hbm.at[pl.ds(off, _K)], didx_v)

            @pl.when(cid == 0)
            def _():
                pltpu.async_copy(ya_hbm.at[sidx_v], rows_v, sem).wait()

            @pl.when(cid == 1)
            def _():
                pltpu.async_copy(yb_hbm.at[sidx_v], rows_v, sem).wait()

            pltpu.sync_copy(rows_v, acc.at[didx_v], add=True)
            if with_info:
                @pl.when(cid == 0)
                def _():
                    pltpu.async_copy(info_hbm.at[sidx_v], info_v, sem).wait()
                    pltpu.sync_copy(info_v, isrc_hbm.at[pl.ds(off, _K)])

                @pl.when(cid == 1)
                def _():
                    pltpu.sync_copy(dst_g_hbm.at[pl.ds(off, _K)], gidx_v)
                    pltpu.async_copy(info_hbm.at[gidx_v], info_v, sem).wait()
                    pltpu.sync_copy(info_v, idst_hbm.at[pl.ds(off, _K)])
        plsc.subcore_barrier()

        @pl.when(cid == 0)
        def _():
            pltpu.sync_copy(acc.at[pl.ds(sid * opt, opt)],
                            agga_hbm.at[pl.ds(sid * opt, opt)])

        @pl.when(cid == 1)
        def _():
            pltpu.sync_copy(acc.at[pl.ds(sid * opt, opt)],
                            aggb_hbm.at[pl.ds(sid * opt, opt)])

    return agg_kernel(ya, yb, src_g, dst_g, dst_s, nodeinfo, zrows)


def _tc_main(x16, deg_col, theta2d, r2d, w1p, b1, w2, b2, wf, bf, n, n2d):
    nb = n // _R
    rb = n2d // nb

    def body(x_ref, deg_ref, th_ref, r_ref, w1_ref, b1_ref, w2_ref, b2_ref,
             wf_ref, bf_ref, ya_ref, yb_ref, dinv_ref, px_ref, py_ref):
        h1 = jnp.maximum(
            jnp.dot(x_ref[...], w1_ref[...],
                    preferred_element_type=jnp.float32) + b1_ref[...], 0.0)
        h2 = jnp.maximum(
            jnp.dot(h1.astype(jnp.bfloat16), w2_ref[...],
                    preferred_element_type=jnp.float32) + b2_ref[...], 0.0)
        xw = jnp.dot(h2.astype(jnp.bfloat16), wf_ref[...],
                     preferred_element_type=jnp.float32) + bf_ref[...]
        dinv = lax.rsqrt(deg_ref[...] + 1.0)
        dinv_ref[...] = dinv
        y1 = dinv * xw
        ya_ref[...] = y1[:, :32]
        yb_ref[...] = y1[:, 32:]
        px_ref[...] = r_ref[...] * jnp.cos(th_ref[...])
        py_ref[...] = r_ref[...] * jnp.sin(th_ref[...])

    return pl.pallas_call(
        body,
        grid=(nb,),
        in_specs=[
            pl.BlockSpec((_R, 16), lambda i: (i, 0)),
            pl.BlockSpec((_R, 1), lambda i: (i, 0)),
            pl.BlockSpec((rb, 128), lambda i: (i, 0)),
            pl.BlockSpec((rb, 128), lambda i: (i, 0)),
            pl.BlockSpec((16, 288), lambda i: (0, 0)),
            pl.BlockSpec((1, 288), lambda i: (0, 0)),
            pl.BlockSpec((288, 1024), lambda i: (0, 0)),
            pl.BlockSpec((1, 1024), lambda i: (0, 0)),
            pl.BlockSpec((1024, 64), lambda i: (0, 0)),
            pl.BlockSpec((1, 64), lambda i: (0, 0)),
        ],
        out_specs=[
            pl.BlockSpec((_R, 32), lambda i: (i, 0)),
            pl.BlockSpec((_R, 32), lambda i: (i, 0)),
            pl.BlockSpec((_R, 1), lambda i: (i, 0)),
            pl.BlockSpec((rb, 128), lambda i: (i, 0)),
            pl.BlockSpec((rb, 128), lambda i: (i, 0)),
        ],
        out_shape=[
            jax.ShapeDtypeStruct((n, 32), jnp.float32),
            jax.ShapeDtypeStruct((n, 32), jnp.float32),
            jax.ShapeDtypeStruct((n, 1), jnp.float32),
            jax.ShapeDtypeStruct((n2d, 128), jnp.float32),
            jax.ShapeDtypeStruct((n2d, 128), jnp.float32),
        ],
    )(x16, deg_col, theta2d, r2d, w1p, b1.reshape(1, 288),
      w2.astype(jnp.bfloat16), b2.reshape(1, 1024), wf.astype(jnp.bfloat16),
      bf.reshape(1, 64))


def _tc_edge(fs, cs, pxs, pys, fd, pxd, pyd, n2d):
    nb = 25
    rb = n2d // nb

    def body(fs_ref, cs_ref, pxs_ref, pys_ref, fd_ref, pxd_ref, pyd_ref,
             a1_ref, a2_ref):
        df = jnp.abs(fs_ref[...] - fd_ref[...])
        a1_ref[...] = jnp.where(df == 1.0,
                                1.0 / jnp.maximum(cs_ref[...], 1.0), 0.0)
        d2 = (pxd_ref[...] - pxs_ref[...]) ** 2 + \
             (pyd_ref[...] - pys_ref[...]) ** 2
        disp = jnp.sqrt(jnp.maximum(d2, 1e-12))
        vel = disp / jnp.where(df == 2.0, df, 1.0)
        a2_ref[...] = jnp.where(df == 2.0, jnp.exp(-vel / 8.5), 0.0)

    spec = pl.BlockSpec((rb, 128), lambda i: (i, 0))
    return pl.pallas_call(
        body,
        grid=(nb,),
        in_specs=[spec] * 7,
        out_specs=[spec] * 2,
        out_shape=[jax.ShapeDtypeStruct((n2d, 128), jnp.float32)] * 2,
    )(fs, cs, pxs, pys, fd, pxd, pyd)


def _tc_mid(agg1a, agg1b, ya, yb, dinv_col, a1_col, gcn1_b, gcn2_w, n):
    nb = n // _R

    def body(aggA_ref, aggB_ref, ya_ref, yb_ref, dinv_ref, a1_ref,
             b1_ref, w2_ref, y2a_ref, y2b_ref):
        dinv = dinv_ref[...]
        agg = jnp.concatenate([aggA_ref[...], aggB_ref[...]], axis=1)
        y1 = jnp.concatenate([ya_ref[...], yb_ref[...]], axis=1)
        g1 = dinv * (agg + y1) + b1_ref[...]
        x1 = jnp.maximum(g1 * a1_ref[...], 0.0)
        y2 = dinv * jnp.dot(x1, w2_ref[...], precision=_HI)
        y2a_ref[...] = y2[:, :32]
        y2b_ref[...] = y2[:, 32:]

    return pl.pallas_call(
        body,
        grid=(nb,),
        in_specs=[
            pl.BlockSpec((_R, 32), lambda i: (i, 0)),
            pl.BlockSpec((_R, 32), lambda i: (i, 0)),
            pl.BlockSpec((_R, 32), lambda i: (i, 0)),
            pl.BlockSpec((_R, 32), lambda i: (i, 0)),
            pl.BlockSpec((_R, 1), lambda i: (i, 0)),
            pl.BlockSpec((_R, 1), lambda i: (i, 0)),
            pl.BlockSpec((1, 64), lambda i: (0, 0)),
            pl.BlockSpec((64, 64), lambda i: (0, 0)),
        ],
        out_specs=[
            pl.BlockSpec((_R, 32), lambda i: (i, 0)),
            pl.BlockSpec((_R, 32), lambda i: (i, 0)),
        ],
        out_shape=[
            jax.ShapeDtypeStruct((n, 32), jnp.float32),
            jax.ShapeDtypeStruct((n, 32), jnp.float32),
        ],
    )(agg1a, agg1b, ya, yb, dinv_col, a1_col,
      gcn1_b.reshape(1, 64), gcn2_w)


def _tc_head(agg2a, agg2b, y2a, y2b, dinv_col, a2_col, gcn2_b, fc_w, fc_b,
             out_w, out_b, n):
    nb = n // _R

    def body(aggA_ref, aggB_ref, ya_ref, yb_ref, dinv_ref, a2_ref, b2_ref,
             fw_ref, fb_ref, ow_ref, ob_ref, o_ref):
        dinv = dinv_ref[...]
        agg = jnp.concatenate([aggA_ref[...], aggB_ref[...]], axis=1)
        y2 = jnp.concatenate([ya_ref[...], yb_ref[...]], axis=1)
        g2 = dinv * (agg + y2) + b2_ref[...]
        x2 = jnp.maximum(g2 * a2_ref[...], 0.0)
        h = jnp.maximum(jnp.dot(x2, fw_ref[...], precision=_HI) + fb_ref[...],
                        0.0)
        logit = jnp.dot(h, ow_ref[...], precision=_HI) + ob_ref[...]
        o_ref[...] = 1.0 / (1.0 + jnp.exp(-logit))

    return pl.pallas_call(
        body,
        grid=(nb,),
        in_specs=[
            pl.BlockSpec((_R, 32), lambda i: (i, 0)),
            pl.BlockSpec((_R, 32), lambda i: (i, 0)),
            pl.BlockSpec((_R, 32), lambda i: (i, 0)),
            pl.BlockSpec((_R, 32), lambda i: (i, 0)),
            pl.BlockSpec((_R, 1), lambda i: (i, 0)),
            pl.BlockSpec((_R, 1), lambda i: (i, 0)),
            pl.BlockSpec((1, 64), lambda i: (0, 0)),
            pl.BlockSpec((64, 32), lambda i: (0, 0)),
            pl.BlockSpec((1, 32), lambda i: (0, 0)),
            pl.BlockSpec((32, 1), lambda i: (0, 0)),
            pl.BlockSpec((1, 1), lambda i: (0, 0)),
        ],
        out_specs=pl.BlockSpec((_R, 1), lambda i: (i, 0)),
        out_shape=jax.ShapeDtypeStruct((n, 1), jnp.float32),
    )(agg2a, agg2b, y2a, y2b, dinv_col, a2_col, gcn2_b.reshape(1, 64), fc_w,
      fc_b.reshape(1, 32), out_w, out_b.reshape(1, 1))


def kernel(x, edge_index, aux_features, conv1_w, conv1_b, conv2_w, conv2_b,
           fcfe_w, fcfe_b, gcn1_w, gcn1_b, gcn2_w, gcn2_b,
           fc_w, fc_b, out_w, out_b):
    n = x.shape[0]
    e = edge_index.shape[1]
    e_pad = -(-e // (_NS * _K)) * (_NS * _K)

    x16 = jnp.zeros((n, 16), jnp.float32).at[:, :9].set(x.reshape(n, 9))
    w1p, b1, w2, b2, wf, bf = _fold_weights(
        conv1_w, conv1_b, conv2_w, conv2_b, fcfe_w, fcfe_b, gcn1_w)
    src = edge_index[0]
    dst = edge_index[1]
    pad = e_pad - e
    pad_g = jnp.zeros((pad,), jnp.int32)
    pad_s = jnp.full((pad,), n, jnp.int32)
    src_g = jnp.concatenate([src, pad_g])
    dst_g = jnp.concatenate([dst, pad_g])
    src_s = jnp.concatenate([src, pad_s])
    dst_s = jnp.concatenate([dst, pad_s])
    scat_idx = jnp.concatenate([dst_s, src_s])
    ones_rows = jnp.zeros((_K, 16), jnp.float32).at[:, 0].set(1.0)
    zpt = _acc_geom(n)[2]
    zrows16 = jnp.zeros((zpt, 16), jnp.float32)
    zrows32 = jnp.zeros((zpt, 32), jnp.float32)

    n2d = e_pad // 128
    npad = e_pad - n

    def to2d(col, fill=0.0):
        return jnp.concatenate(
            [col, jnp.full((npad,), fill, jnp.float32)]).reshape(n2d, 128)

    theta2d = to2d(aux_features[:, 0])
    r2d = to2d(aux_features[:, 1])

    deg16, cnt16 = _sc_s1(scat_idx, ones_rows, zrows16, n, e_pad)
    deg_col = deg16[:n, 0:1]
    ya, yb, dinv_col, px2d, py2d = _tc_main(
        x16, deg_col, theta2d, r2d, w1p, b1, w2, b2, wf, bf, n, n2d)
    px_col = px2d.reshape(e_pad, 1)
    py_col = py2d.reshape(e_pad, 1)
    nodeinfo = jnp.concatenate(
        [aux_features[:, 2:3], cnt16[:n, 0:1], px_col[:n], py_col[:n],
         jnp.zeros((n, 12), jnp.float32)], axis=1)
    agg1a, agg1b, isrc, idst = _sc_agg(ya, yb, src_g, dst_g, dst_s,
                                       nodeinfo, zrows32, n, e_pad,
                                       with_info=True)
    a1_2d, a2_2d = _tc_edge(
        isrc[:, 0].reshape(n2d, 128), isrc[:, 1].reshape(n2d, 128),
        isrc[:, 2].reshape(n2d, 128), isrc[:, 3].reshape(n2d, 128),
        idst[:, 0].reshape(n2d, 128), idst[:, 2].reshape(n2d, 128),
        idst[:, 3].reshape(n2d, 128), n2d)
    a1_col = a1_2d.reshape(e_pad, 1)
    a2_col = a2_2d.reshape(e_pad, 1)
    y2a, y2b = _tc_mid(agg1a, agg1b, ya, yb, dinv_col, a1_col[:n],
                       gcn1_b, gcn2_w, n)
    agg2a, agg2b = _sc_agg(y2a, y2b, src_g, dst_g, dst_s, nodeinfo,
                           zrows32, n, e_pad, with_info=False)
    out = _tc_head(agg2a, agg2b, y2a, y2b, dinv_col, a2_col[:n], gcn2_b,
                   fc_w, fc_b, out_w, out_b, n)
    return out


# speedup vs baseline: 3.6307x; 1.1210x over previous
import functools

import jax
import jax.numpy as jnp
import numpy as np
from jax import lax
from jax.experimental import pallas as pl
from jax.experimental.pallas import tpu as pltpu
from jax.experimental.pallas import tpu_sc as plsc

_NC = 2
_NS = 16
_K = 128
_R = 2000

_HI = lax.Precision.HIGHEST


def _im2col_tables():
    g1 = np.full((16, 288), 288, np.int32)
    for c in range(32):
        for i in range(3):
            for j in range(3):
                for u in range(3):
                    for v in range(3):
                        i1, j1 = i + u - 1, j + v - 1
                        if 0 <= i1 < 3 and 0 <= j1 < 3:
                            g1[i1 * 3 + j1, c * 9 + i * 3 + j] = \
                                (c * 3 + u) * 3 + v
    g2 = np.full((288, 1024), 8192, np.int32)
    for c2 in range(64):
        for c1_ in range(32):
            for i2 in range(4):
                for j2 in range(4):
                    for u in range(2):
                        for v in range(2):
                            i1, j1 = i2 + u - 1, j2 + v - 1
                            if 0 <= i1 < 3 and 0 <= j1 < 3:
                                g2[c1_ * 9 + i1 * 3 + j1,
                                   c2 * 16 + i2 * 4 + j2] = \
                                    ((c2 * 32 + c1_) * 2 + u) * 2 + v
    pool = np.zeros((1024, 256), np.float32)
    for c in range(64):
        for i in range(4):
            for j in range(4):
                pool[c * 16 + i * 4 + j, c * 4 + (i // 2) * 2 + j // 2] = 0.25
    return g1, g2, pool


_G1, _G2, _POOL = _im2col_tables()


def _conv_nb(x, w, pad):
    return lax.conv_general_dilated(
        x, w, (1, 1), [(pad, pad), (pad, pad)],
        dimension_numbers=('NCHW', 'OIHW', 'NCHW'))


def _fold_weights(conv1_w, conv1_b, conv2_w, conv2_b, fcfe_w, fcfe_b, gcn1_w):
    eye9 = jnp.eye(9, dtype=jnp.float32).reshape(9, 1, 3, 3)
    w1 = _conv_nb(eye9, conv1_w, 1).reshape(9, 288)
    w1p = jnp.zeros((16, 288), jnp.float32).at[:9].set(w1)
    b1 = jnp.repeat(conv1_b, 9)
    eye288 = jnp.eye(288, dtype=jnp.float32).reshape(288, 32, 3, 3)
    w2 = _conv_nb(eye288, conv2_w, 1).reshape(288, 1024)
    b2 = jnp.repeat(conv2_b, 16)
    wf = jnp.asarray(_POOL) @ (fcfe_w @ gcn1_w)
    bf = fcfe_b @ gcn1_w
    return w1p, b1, w2, b2, wf, bf


def _acc_geom(n):
    opt = -(-(-(-n // _NS)) // 8) * 8
    n_out = _NS * opt
    zpt = -(-max(n_out, n + 1) // _NS // 8) * 8
    n_acc = _NS * zpt
    return opt, n_out, zpt, n_acc


def _sc_s1(scat_idx, ones_rows, zrows, n, e_pad):
    opt, n_out, zpt, n_acc = _acc_geom(n)
    chunks = e_pad // (_NS * _K)
    mesh = plsc.VectorSubcoreMesh(core_axis_name="c", subcore_axis_name="s",
                                  num_cores=_NC, num_subcores=_NS)

    @functools.partial(
        pl.kernel, mesh=mesh,
        out_type=[jax.ShapeDtypeStruct((n_out, 16), jnp.float32),
                  jax.ShapeDtypeStruct((n_out, 16), jnp.float32)],
        scratch_types=[
            pltpu.VMEM_SHARED((n_acc, 16), jnp.float32),
            pltpu.VMEM((_K,), jnp.int32),
            pltpu.VMEM((_K, 16), jnp.float32),
        ],
        compiler_params=pltpu.CompilerParams(use_tc_tiling_on_sc=False))
    def s1(scat_idx_hbm, ones_hbm, zrows_hbm, deg_hbm, cnt_hbm,
           acc, idx_v, ones_v):
        cid = lax.axis_index("c")
        sid = lax.axis_index("s")
        pltpu.sync_copy(zrows_hbm, acc.at[pl.ds(sid * zpt, zpt)])
        pltpu.sync_copy(ones_hbm, ones_v)
        plsc.subcore_barrier()
        for j in range(chunks):
            off = cid * e_pad + sid * (chunks * _K) + j * _K
            pltpu.sync_copy(scat_idx_hbm.at[pl.ds(off, _K)], idx_v)
            pltpu.sync_copy(ones_v, acc.at[idx_v], add=True)
        plsc.subcore_barrier()

        @pl.when(cid == 0)
        def _():
            pltpu.sync_copy(acc.at[pl.ds(sid * opt, opt)],
                            deg_hbm.at[pl.ds(sid * opt, opt)])

        @pl.when(cid == 1)
        def _():
            pltpu.sync_copy(acc.at[pl.ds(sid * opt, opt)],
                            cnt_hbm.at[pl.ds(sid * opt, opt)])

    return s1(scat_idx, ones_rows, zrows)


def _sc_agg(ya, yb, src_g, dst_g, dst_s, nodeinfo, zrows, n, e_pad, with_info):
    opt, n_out, zpt, n_acc = _acc_geom(n)
    chunks = e_pad // (_NS * _K)
    mesh = plsc.VectorSubcoreMesh(core_axis_name="c", subcore_axis_name="s",
                                  num_cores=_NC, num_subcores=_NS)
    out_type = [jax.ShapeDtypeStruct((n_out, 32), jnp.float32),
                jax.ShapeDtypeStruct((n_out, 32), jnp.float32)]
    if with_info:
        out_type += [jax.ShapeDtypeStruct((e_pad, 16), jnp.float32),
                     jax.ShapeDtypeStruct((e_pad, 16), jnp.float32)]

    @functools.partial(
        pl.kernel, mesh=mesh,
        out_type=out_type,
        scratch_types=[
            pltpu.VMEM_SHARED((n_acc, 32), jnp.float32),
            pltpu.VMEM((_K,), jnp.int32),
            pltpu.VMEM((_K,), jnp.int32),
            pltpu.VMEM((_K,), jnp.int32),
            pltpu.VMEM((_K, 32), jnp.float32),
            pltpu.VMEM((_K, 16), jnp.float32),
            pltpu.SemaphoreType.DMA,
        ],
        compiler_params=pltpu.CompilerParams(use_tc_tiling_on_sc=False))
    def agg_kernel(ya_hbm, yb_hbm, src_g_hbm, dst_g_hbm, dst_s_hbm,
                   info_hbm, zrows_hbm, *out_and_scratch):
        if with_info:
            agga_hbm, aggb_hbm, isrc_hbm, idst_hbm = out_and_scratch[:4]
            rest = out_and_scratch[4:]
        else:
            agga_hbm, aggb_hbm = out_and_scratch[:2]
            rest = out_and_scratch[2:]
        acc, sidx_v, didx_v, gidx_v, rows_v, info_v, sem = rest
        cid = lax.axis_index("c")
        sid = lax.axis_index("s")
        pltpu.sync_copy(zrows_hbm, acc.at[pl.ds(sid * zpt, zpt)])
        plsc.subcore_barrier()
        for j in range(chunks):
            off = sid * (chunks * _K) + j * _K
            pltpu.sync_copy(src_g_hbm.at[pl.ds(off, _K)], sidx_v)
            pltpu.sync_copy(dst_s_hbm.at[pl.ds(off, _K)], didx_v)

            @pl.when(cid == 0)
            def _():
                pltpu.async_copy(ya_hbm.at[sidx_v], rows_v, sem).wait()

            @pl.when(cid == 1)
            def _():
                pltpu.async_copy(yb_hbm.at[sidx_v], rows_v, sem).wait()

            pltpu.sync_copy(rows_v, acc.at[didx_v], add=True)
            if with_info:
                @pl.when(cid == 0)
                def _():
                    pltpu.async_copy(info_hbm.at[sidx_v], info_v, sem).wait()
                    pltpu.sync_copy(info_v, isrc_hbm.at[pl.ds(off, _K)])

                @pl.when(cid == 1)
                def _():
                    pltpu.sync_copy(dst_g_hbm.at[pl.ds(off, _K)], gidx_v)
                    pltpu.async_copy(info_hbm.at[gidx_v], info_v, sem).wait()
                    pltpu.sync_copy(info_v, idst_hbm.at[pl.ds(off, _K)])
        plsc.subcore_barrier()

        @pl.when(cid == 0)
        def _():
            pltpu.sync_copy(acc.at[pl.ds(sid * opt, opt)],
                            agga_hbm.at[pl.ds(sid * opt, opt)])

        @pl.when(cid == 1)
        def _():
            pltpu.sync_copy(acc.at[pl.ds(sid * opt, opt)],
                            aggb_hbm.at[pl.ds(sid * opt, opt)])

    return agg_kernel(ya, yb, src_g, dst_g, dst_s, nodeinfo, zrows)


def _tc_main(x16, deg_col, theta2d, r2d, w1p, b1, w2, b2, wf, bf, n, n2d):
    nb = n // _R
    rb = n2d // nb

    def body(x_ref, deg_ref, th_ref, r_ref, w1_ref, b1_ref, w2_ref, b2_ref,
             wf_ref, bf_ref, ya_ref, yb_ref, dinv_ref, px_ref, py_ref):
        h1 = jnp.maximum(
            jnp.dot(x_ref[...], w1_ref[...],
                    preferred_element_type=jnp.float32) + b1_ref[...], 0.0)
        h2 = jnp.maximum(
            jnp.dot(h1.astype(jnp.bfloat16), w2_ref[...],
                    preferred_element_type=jnp.float32) + b2_ref[...], 0.0)
        xw = jnp.dot(h2.astype(jnp.bfloat16), wf_ref[...],
                     preferred_element_type=jnp.float32) + bf_ref[...]
        dinv = lax.rsqrt(deg_ref[...] + 1.0)
        dinv_ref[...] = dinv
        y1 = dinv * xw
        ya_ref[...] = y1[:, :32]
        yb_ref[...] = y1[:, 32:]
        px_ref[...] = r_ref[...] * jnp.cos(th_ref[...])
        py_ref[...] = r_ref[...] * jnp.sin(th_ref[...])

    return pl.pallas_call(
        body,
        grid=(nb,),
        in_specs=[
            pl.BlockSpec((_R, 16), lambda i: (i, 0)),
            pl.BlockSpec((_R, 1), lambda i: (i, 0)),
            pl.BlockSpec((rb, 128), lambda i: (i, 0)),
            pl.BlockSpec((rb, 128), lambda i: (i, 0)),
            pl.BlockSpec((16, 288), lambda i: (0, 0)),
            pl.BlockSpec((1, 288), lambda i: (0, 0)),
            pl.BlockSpec((288, 1024), lambda i: (0, 0)),
            pl.BlockSpec((1, 1024), lambda i: (0, 0)),
            pl.BlockSpec((1024, 64), lambda i: (0, 0)),
            pl.BlockSpec((1, 64), lambda i: (0, 0)),
        ],
        out_specs=[
            pl.BlockSpec((_R, 32), lambda i: (i, 0)),
            pl.BlockSpec((_R, 32), lambda i: (i, 0)),
            pl.BlockSpec((_R, 1), lambda i: (i, 0)),
            pl.BlockSpec((rb, 128), lambda i: (i, 0)),
            pl.BlockSpec((rb, 128), lambda i: (i, 0)),
        ],
        out_shape=[
            jax.ShapeDtypeStruct((n, 32), jnp.float32),
            jax.ShapeDtypeStruct((n, 32), jnp.float32),
            jax.ShapeDtypeStruct((n, 1), jnp.float32),
            jax.ShapeDtypeStruct((n2d, 128), jnp.float32),
            jax.ShapeDtypeStruct((n2d, 128), jnp.float32),
        ],
    )(x16, deg_col, theta2d, r2d, w1p, b1.reshape(1, 288),
      w2.astype(jnp.bfloat16), b2.reshape(1, 1024), wf.astype(jnp.bfloat16),
      bf.reshape(1, 64))


def _tc_edge(fs, cs, pxs, pys, fd, pxd, pyd, n2d):
    nb = 25
    rb = n2d // nb

    def body(fs_ref, cs_ref, pxs_ref, pys_ref, fd_ref, pxd_ref, pyd_ref,
             a1_ref, a2_ref):
        df = jnp.abs(fs_ref[...] - fd_ref[...])
        a1_ref[...] = jnp.where(df == 1.0,
                                1.0 / jnp.maximum(cs_ref[...], 1.0), 0.0)
        d2 = (pxd_ref[...] - pxs_ref[...]) ** 2 + \
             (pyd_ref[...] - pys_ref[...]) ** 2
        disp = jnp.sqrt(jnp.maximum(d2, 1e-12))
        vel = disp / jnp.where(df == 2.0, df, 1.0)
        a2_ref[...] = jnp.where(df == 2.0, jnp.exp(-vel / 8.5), 0.0)

    spec = pl.BlockSpec((rb, 128), lambda i: (i, 0))
    return pl.pallas_call(
        body,
        grid=(nb,),
        in_specs=[spec] * 7,
        out_specs=[spec] * 2,
        out_shape=[jax.ShapeDtypeStruct((n2d, 128), jnp.float32)] * 2,
    )(fs, cs, pxs, pys, fd, pxd, pyd)


def _tc_mid(agg1a, agg1b, ya, yb, dinv_col, isrc, idst, gcn1_b, gcn2_w, n):
    nb = n // _R

    def body(aggA_ref, aggB_ref, ya_ref, yb_ref, dinv_ref, is_ref, id_ref,
             b1_ref, w2_ref, y2a_ref, y2b_ref, a2_ref):
        dinv = dinv_ref[...]
        agg = jnp.concatenate([aggA_ref[...], aggB_ref[...]], axis=1)
        y1 = jnp.concatenate([ya_ref[...], yb_ref[...]], axis=1)
        g1 = dinv * (agg + y1) + b1_ref[...]
        fs = is_ref[:, 0:1]
        cs = is_ref[:, 1:2]
        pxs = is_ref[:, 2:3]
        pys = is_ref[:, 3:4]
        fd = id_ref[:, 0:1]
        pxd = id_ref[:, 2:3]
        pyd = id_ref[:, 3:4]
        df = jnp.abs(fs - fd)
        a1 = jnp.where(df == 1.0, 1.0 / jnp.maximum(cs, 1.0), 0.0)
        x1 = jnp.maximum(g1 * a1, 0.0)
        d2 = (pxd - pxs) ** 2 + (pyd - pys) ** 2
        disp = jnp.sqrt(jnp.maximum(d2, 1e-12))
        vel = disp / jnp.where(df == 2.0, df, 1.0)
        a2_ref[...] = jnp.where(df == 2.0, jnp.exp(-vel / 8.5), 0.0)
        y2 = dinv * jnp.dot(x1, w2_ref[...], precision=_HI)
        y2a_ref[...] = y2[:, :32]
        y2b_ref[...] = y2[:, 32:]

    return pl.pallas_call(
        body,
        grid=(nb,),
        in_specs=[
            pl.BlockSpec((_R, 32), lambda i: (i, 0)),
            pl.BlockSpec((_R, 32), lambda i: (i, 0)),
            pl.BlockSpec((_R, 32), lambda i: (i, 0)),
            pl.BlockSpec((_R, 32), lambda i: (i, 0)),
            pl.BlockSpec((_R, 1), lambda i: (i, 0)),
            pl.BlockSpec((_R, 16), lambda i: (i, 0)),
            pl.BlockSpec((_R, 16), lambda i: (i, 0)),
            pl.BlockSpec((1, 64), lambda i: (0, 0)),
            pl.BlockSpec((64, 64), lambda i: (0, 0)),
        ],
        out_specs=[
            pl.BlockSpec((_R, 32), lambda i: (i, 0)),
            pl.BlockSpec((_R, 32), lambda i: (i, 0)),
            pl.BlockSpec((_R, 1), lambda i: (i, 0)),
        ],
        out_shape=[
            jax.ShapeDtypeStruct((n, 32), jnp.float32),
            jax.ShapeDtypeStruct((n, 32), jnp.float32),
            jax.ShapeDtypeStruct((n, 1), jnp.float32),
        ],
    )(agg1a, agg1b, ya, yb, dinv_col, isrc, idst,
      gcn1_b.reshape(1, 64), gcn2_w)


def _tc_head(agg2a, agg2b, y2a, y2b, dinv_col, a2_col, gcn2_b, fc_w, fc_b,
             out_w, out_b, n):
    nb = n // _R

    def body(aggA_ref, aggB_ref, ya_ref, yb_ref, dinv_ref, a2_ref, b2_ref,
             fw_ref, fb_ref, ow_ref, ob_ref, o_ref):
        dinv = dinv_ref[...]
        agg = jnp.concatenate([aggA_ref[...], aggB_ref[...]], axis=1)
        y2 = jnp.concatenate([ya_ref[...], yb_ref[...]], axis=1)
        g2 = dinv * (agg + y2) + b2_ref[...]
        x2 = jnp.maximum(g2 * a2_ref[...], 0.0)
        h = jnp.maximum(jnp.dot(x2, fw_ref[...], precision=_HI) + fb_ref[...],
                        0.0)
        logit = jnp.dot(h, ow_ref[...], precision=_HI) + ob_ref[...]
        o_ref[...] = 1.0 / (1.0 + jnp.exp(-logit))

    return pl.pallas_call(
        body,
        grid=(nb,),
        in_specs=[
            pl.BlockSpec((_R, 32), lambda i: (i, 0)),
            pl.BlockSpec((_R, 32), lambda i: (i, 0)),
            pl.BlockSpec((_R, 32), lambda i: (i, 0)),
            pl.BlockSpec((_R, 32), lambda i: (i, 0)),
            pl.BlockSpec((_R, 1), lambda i: (i, 0)),
            pl.BlockSpec((_R, 1), lambda i: (i, 0)),
            pl.BlockSpec((1, 64), lambda i: (0, 0)),
            pl.BlockSpec((64, 32), lambda i: (0, 0)),
            pl.BlockSpec((1, 32), lambda i: (0, 0)),
            pl.BlockSpec((32, 1), lambda i: (0, 0)),
            pl.BlockSpec((1, 1), lambda i: (0, 0)),
        ],
        out_specs=pl.BlockSpec((_R, 1), lambda i: (i, 0)),
        out_shape=jax.ShapeDtypeStruct((n, 1), jnp.float32),
    )(agg2a, agg2b, y2a, y2b, dinv_col, a2_col, gcn2_b.reshape(1, 64), fc_w,
      fc_b.reshape(1, 32), out_w, out_b.reshape(1, 1))


def kernel(x, edge_index, aux_features, conv1_w, conv1_b, conv2_w, conv2_b,
           fcfe_w, fcfe_b, gcn1_w, gcn1_b, gcn2_w, gcn2_b,
           fc_w, fc_b, out_w, out_b):
    n = x.shape[0]
    e = edge_index.shape[1]
    e_pad = -(-e // (_NS * _K)) * (_NS * _K)

    x16 = jnp.zeros((n, 16), jnp.float32).at[:, :9].set(x.reshape(n, 9))
    w1p, b1, w2, b2, wf, bf = _fold_weights(
        conv1_w, conv1_b, conv2_w, conv2_b, fcfe_w, fcfe_b, gcn1_w)
    src = edge_index[0]
    dst = edge_index[1]
    pad = e_pad - e
    pad_g = jnp.zeros((pad,), jnp.int32)
    pad_s = jnp.full((pad,), n, jnp.int32)
    src_g = jnp.concatenate([src, pad_g])
    dst_g = jnp.concatenate([dst, pad_g])
    src_s = jnp.concatenate([src, pad_s])
    dst_s = jnp.concatenate([dst, pad_s])
    scat_idx = jnp.concatenate([dst_s, src_s])
    ones_rows = jnp.zeros((_K, 16), jnp.float32).at[:, 0].set(1.0)
    zpt = _acc_geom(n)[2]
    zrows16 = jnp.zeros((zpt, 16), jnp.float32)
    zrows32 = jnp.zeros((zpt, 32), jnp.float32)

    n2d = e_pad // 128
    npad = e_pad - n

    def to2d(col, fill=0.0):
        return jnp.concatenate(
            [col, jnp.full((npad,), fill, jnp.float32)]).reshape(n2d, 128)

    theta2d = to2d(aux_features[:, 0])
    r2d = to2d(aux_features[:, 1])

    deg16, cnt16 = _sc_s1(scat_idx, ones_rows, zrows16, n, e_pad)
    deg_col = deg16[:n, 0:1]
    ya, yb, dinv_col, px2d, py2d = _tc_main(
        x16, deg_col, theta2d, r2d, w1p, b1, w2, b2, wf, bf, n, n2d)
    px_col = px2d.reshape(e_pad, 1)
    py_col = py2d.reshape(e_pad, 1)
    nodeinfo = jnp.concatenate(
        [aux_features[:, 2:3], cnt16[:n, 0:1], px_col[:n], py_col[:n],
         jnp.zeros((n, 12), jnp.float32)], axis=1)
    agg1a, agg1b, isrc, idst = _sc_agg(ya, yb, src_g, dst_g, dst_s,
                                       nodeinfo, zrows32, n, e_pad,
                                       with_info=True)
    y2a, y2b, a2_col = _tc_mid(agg1a, agg1b, ya, yb, dinv_col, isrc, idst,
                               gcn1_b, gcn2_w, n)
    agg2a, agg2b = _sc_agg(y2a, y2b, src_g, dst_g, dst_s, nodeinfo,
                           zrows32, n, e_pad, with_info=False)
    out = _tc_head(agg2a, agg2b, y2a, y2b, dinv_col, a2_col, gcn2_b,
                   fc_w, fc_b, out_w, out_b, n)
    return out


# speedup vs baseline: 3.9857x; 1.0978x over previous
import functools

import jax
import jax.numpy as jnp
import numpy as np
from jax import lax
from jax.experimental import pallas as pl
from jax.experimental.pallas import tpu as pltpu
from jax.experimental.pallas import tpu_sc as plsc

_NC = 2
_NS = 16
_K = 128
_R = 2000

_HI = lax.Precision.HIGHEST


def _im2col_tables():
    g1 = np.full((16, 288), 288, np.int32)
    for c in range(32):
        for i in range(3):
            for j in range(3):
                for u in range(3):
                    for v in range(3):
                        i1, j1 = i + u - 1, j + v - 1
                        if 0 <= i1 < 3 and 0 <= j1 < 3:
                            g1[i1 * 3 + j1, c * 9 + i * 3 + j] = \
                                (c * 3 + u) * 3 + v
    g2 = np.full((288, 1024), 8192, np.int32)
    for c2 in range(64):
        for c1_ in range(32):
            for i2 in range(4):
                for j2 in range(4):
                    for u in range(2):
                        for v in range(2):
                            i1, j1 = i2 + u - 1, j2 + v - 1
                            if 0 <= i1 < 3 and 0 <= j1 < 3:
                                g2[c1_ * 9 + i1 * 3 + j1,
                                   c2 * 16 + i2 * 4 + j2] = \
                                    ((c2 * 32 + c1_) * 2 + u) * 2 + v
    pool = np.zeros((1024, 256), np.float32)
    for c in range(64):
        for i in range(4):
            for j in range(4):
                pool[c * 16 + i * 4 + j, c * 4 + (i // 2) * 2 + j // 2] = 0.25
    return g1, g2, pool


_G1, _G2, _POOL = _im2col_tables()


def _conv_nb(x, w, pad):
    return lax.conv_general_dilated(
        x, w, (1, 1), [(pad, pad), (pad, pad)],
        dimension_numbers=('NCHW', 'OIHW', 'NCHW'))


def _fold_weights(conv1_w, conv1_b, conv2_w, conv2_b, fcfe_w, fcfe_b, gcn1_w):
    eye9 = jnp.eye(9, dtype=jnp.float32).reshape(9, 1, 3, 3)
    w1 = _conv_nb(eye9, conv1_w, 1).reshape(9, 288)
    w1p = jnp.zeros((16, 288), jnp.float32).at[:9].set(w1)
    b1 = jnp.repeat(conv1_b, 9)
    eye288 = jnp.eye(288, dtype=jnp.float32).reshape(288, 32, 3, 3)
    w2 = _conv_nb(eye288, conv2_w, 1).reshape(288, 1024)
    b2 = jnp.repeat(conv2_b, 16)
    wf = jnp.asarray(_POOL) @ (fcfe_w @ gcn1_w)
    bf = fcfe_b @ gcn1_w
    return w1p, b1, w2, b2, wf, bf


def _acc_geom(n):
    opt = -(-(-(-n // _NS)) // 8) * 8
    n_out = _NS * opt
    zpt = -(-max(n_out, n + 1) // _NS // 8) * 8
    n_acc = _NS * zpt
    return opt, n_out, zpt, n_acc


def _sc_s1(scat_idx, ones_rows, zrows, n, e_pad):
    opt, n_out, zpt, n_acc = _acc_geom(n)
    chunks = e_pad // (_NS * _K)
    mesh = plsc.VectorSubcoreMesh(core_axis_name="c", subcore_axis_name="s",
                                  num_cores=_NC, num_subcores=_NS)

    @functools.partial(
        pl.kernel, mesh=mesh,
        out_type=[jax.ShapeDtypeStruct((n_out, 16), jnp.float32),
                  jax.ShapeDtypeStruct((n_out, 16), jnp.float32)],
        scratch_types=[
            pltpu.VMEM_SHARED((n_acc, 16), jnp.float32),
            pltpu.VMEM((chunks, _K), jnp.int32),
            pltpu.VMEM((_K, 16), jnp.float32),
            pltpu.SemaphoreType.DMA,
            pltpu.SemaphoreType.DMA,
        ],
        compiler_params=pltpu.CompilerParams(use_tc_tiling_on_sc=False))
    def s1(scat_idx_hbm, ones_hbm, zrows_hbm, deg_hbm, cnt_hbm,
           acc, idx_v, ones_v, isem, ssem):
        cid = lax.axis_index("c")
        sid = lax.axis_index("s")
        base = cid * e_pad + sid * (chunks * _K)
        idesc = [pltpu.async_copy(
            scat_idx_hbm.at[pl.ds(base + j * _K, _K)], idx_v.at[j], isem)
            for j in range(chunks)]
        pltpu.sync_copy(zrows_hbm, acc.at[pl.ds(sid * zpt, zpt)])
        pltpu.sync_copy(ones_hbm, ones_v)
        plsc.subcore_barrier()
        for d in idesc:
            d.wait()
        sdesc = [pltpu.async_copy(ones_v, acc.at[idx_v.at[j]], ssem,
                                  add=True) for j in range(chunks)]
        for d in sdesc:
            d.wait()
        plsc.subcore_barrier()

        @pl.when(cid == 0)
        def _():
            pltpu.sync_copy(acc.at[pl.ds(sid * opt, opt)],
                            deg_hbm.at[pl.ds(sid * opt, opt)])

        @pl.when(cid == 1)
        def _():
            pltpu.sync_copy(acc.at[pl.ds(sid * opt, opt)],
                            cnt_hbm.at[pl.ds(sid * opt, opt)])

    return s1(scat_idx, ones_rows, zrows)


def _sc_agg(ya, yb, src_g, dst_g, dst_s, nodeinfo, zrows, n, e_pad, with_info):
    opt, n_out, zpt, n_acc = _acc_geom(n)
    chunks = e_pad // (_NS * _K)
    mesh = plsc.VectorSubcoreMesh(core_axis_name="c", subcore_axis_name="s",
                                  num_cores=_NC, num_subcores=_NS)
    out_type = [jax.ShapeDtypeStruct((n_out, 32), jnp.float32),
                jax.ShapeDtypeStruct((n_out, 32), jnp.float32)]
    if with_info:
        out_type += [jax.ShapeDtypeStruct((e_pad, 16), jnp.float32),
                     jax.ShapeDtypeStruct((e_pad, 16), jnp.float32)]

    nbuf = 4
    nbuf_i = 2

    @functools.partial(
        pl.kernel, mesh=mesh,
        out_type=out_type,
        scratch_types=[
            pltpu.VMEM_SHARED((n_acc, 32), jnp.float32),
            pltpu.VMEM((chunks, _K), jnp.int32),
            pltpu.VMEM((chunks, _K), jnp.int32),
            pltpu.VMEM((chunks, _K), jnp.int32),
            pltpu.VMEM((nbuf, _K, 32), jnp.float32),
            pltpu.VMEM((nbuf_i, _K, 16), jnp.float32),
            pltpu.SemaphoreType.DMA,
            pltpu.SemaphoreType.DMA,
            pltpu.SemaphoreType.DMA,
            pltpu.SemaphoreType.DMA,
            pltpu.SemaphoreType.DMA,
        ],
        compiler_params=pltpu.CompilerParams(use_tc_tiling_on_sc=False))
    def agg_kernel(ya_hbm, yb_hbm, src_g_hbm, dst_g_hbm, dst_s_hbm,
                   info_hbm, zrows_hbm, *out_and_scratch):
        if with_info:
            agga_hbm, aggb_hbm, isrc_hbm, idst_hbm = out_and_scratch[:4]
            rest = out_and_scratch[4:]
        else:
            agga_hbm, aggb_hbm = out_and_scratch[:2]
            isrc_hbm = idst_hbm = None
            rest = out_and_scratch[2:]
        acc, sidx, didx, gidx, rows, ibuf, isem, gsem, ssem, igsem, iwsem \
            = rest
        cid = lax.axis_index("c")
        sid = lax.axis_index("s")
        base = sid * (chunks * _K)
        idesc = []
        for j in range(chunks):
            idesc.append(pltpu.async_copy(
                src_g_hbm.at[pl.ds(base + j * _K, _K)], sidx.at[j], isem))
            idesc.append(pltpu.async_copy(
                dst_s_hbm.at[pl.ds(base + j * _K, _K)], didx.at[j], isem))
            if with_info:
                idesc.append(pltpu.async_copy(
                    dst_g_hbm.at[pl.ds(base + j * _K, _K)], gidx.at[j],
                    isem))
        pltpu.sync_copy(zrows_hbm, acc.at[pl.ds(sid * zpt, zpt)])
        plsc.subcore_barrier()
        for d in idesc:
            d.wait()

        def run(y_hbm, g_all, iout_hbm):
            gdesc = [None] * chunks
            sdesc = [None] * chunks
            igdesc = [None] * chunks
            iwdesc = [None] * chunks
            gdesc[0] = pltpu.async_copy(y_hbm.at[sidx.at[0]], rows.at[0],
                                        gsem)
            if with_info:
                igdesc[0] = pltpu.async_copy(info_hbm.at[g_all.at[0]],
                                             ibuf.at[0], igsem)
            for j in range(chunks):
                gdesc[j].wait()
                if j + 1 < chunks:
                    if j + 1 >= nbuf:
                        sdesc[j + 1 - nbuf].wait()
                    gdesc[j + 1] = pltpu.async_copy(
                        y_hbm.at[sidx.at[j + 1]], rows.at[(j + 1) % nbuf],
                        gsem)
                sdesc[j] = pltpu.async_copy(rows.at[j % nbuf],
                                            acc.at[didx.at[j]], ssem,
                                            add=True)
                if with_info:
                    igdesc[j].wait()
                    if j + 1 < chunks:
                        if j + 1 >= nbuf_i:
                            iwdesc[j + 1 - nbuf_i].wait()
                        igdesc[j + 1] = pltpu.async_copy(
                            info_hbm.at[g_all.at[j + 1]],
                            ibuf.at[(j + 1) % nbuf_i], igsem)
                    iwdesc[j] = pltpu.async_copy(
                        ibuf.at[j % nbuf_i],
                        iout_hbm.at[pl.ds(base + j * _K, _K)], iwsem)
            for j in range(max(0, chunks - nbuf), chunks):
                sdesc[j].wait()
            if with_info:
                for j in range(max(0, chunks - nbuf_i), chunks):
                    iwdesc[j].wait()

        @pl.when(cid == 0)
        def _():
            run(ya_hbm, sidx, isrc_hbm)

        @pl.when(cid == 1)
        def _():
            run(yb_hbm, gidx, idst_hbm)

        plsc.subcore_barrier()

        @pl.when(cid == 0)
        def _():
            pltpu.sync_copy(acc.at[pl.ds(sid * opt, opt)],
                            agga_hbm.at[pl.ds(sid * opt, opt)])

        @pl.when(cid == 1)
        def _():
            pltpu.sync_copy(acc.at[pl.ds(sid * opt, opt)],
                            aggb_hbm.at[pl.ds(sid * opt, opt)])

    return agg_kernel(ya, yb, src_g, dst_g, dst_s, nodeinfo, zrows)


def _tc_main(x16, deg_col, theta2d, r2d, w1p, b1, w2, b2, wf, bf, n, n2d):
    nb = n // _R
    rb = n2d // nb

    def body(x_ref, deg_ref, th_ref, r_ref, w1_ref, b1_ref, w2_ref, b2_ref,
             wf_ref, bf_ref, ya_ref, yb_ref, dinv_ref, px_ref, py_ref):
        h1 = jnp.maximum(
            jnp.dot(x_ref[...], w1_ref[...],
                    preferred_element_type=jnp.float32) + b1_ref[...], 0.0)
        h2 = jnp.maximum(
            jnp.dot(h1.astype(jnp.bfloat16), w2_ref[...],
                    preferred_element_type=jnp.float32) + b2_ref[...], 0.0)
        xw = jnp.dot(h2.astype(jnp.bfloat16), wf_ref[...],
                     preferred_element_type=jnp.float32) + bf_ref[...]
        dinv = lax.rsqrt(deg_ref[...] + 1.0)
        dinv_ref[...] = dinv
        y1 = dinv * xw
        ya_ref[...] = y1[:, :32]
        yb_ref[...] = y1[:, 32:]
        px_ref[...] = r_ref[...] * jnp.cos(th_ref[...])
        py_ref[...] = r_ref[...] * jnp.sin(th_ref[...])

    return pl.pallas_call(
        body,
        grid=(nb,),
        in_specs=[
            pl.BlockSpec((_R, 16), lambda i: (i, 0)),
            pl.BlockSpec((_R, 1), lambda i: (i, 0)),
            pl.BlockSpec((rb, 128), lambda i: (i, 0)),
            pl.BlockSpec((rb, 128), lambda i: (i, 0)),
            pl.BlockSpec((16, 288), lambda i: (0, 0)),
            pl.BlockSpec((1, 288), lambda i: (0, 0)),
            pl.BlockSpec((288, 1024), lambda i: (0, 0)),
            pl.BlockSpec((1, 1024), lambda i: (0, 0)),
            pl.BlockSpec((1024, 64), lambda i: (0, 0)),
            pl.BlockSpec((1, 64), lambda i: (0, 0)),
        ],
        out_specs=[
            pl.BlockSpec((_R, 32), lambda i: (i, 0)),
            pl.BlockSpec((_R, 32), lambda i: (i, 0)),
            pl.BlockSpec((_R, 1), lambda i: (i, 0)),
            pl.BlockSpec((rb, 128), lambda i: (i, 0)),
            pl.BlockSpec((rb, 128), lambda i: (i, 0)),
        ],
        out_shape=[
            jax.ShapeDtypeStruct((n, 32), jnp.float32),
            jax.ShapeDtypeStruct((n, 32), jnp.float32),
            jax.ShapeDtypeStruct((n, 1), jnp.float32),
            jax.ShapeDtypeStruct((n2d, 128), jnp.float32),
            jax.ShapeDtypeStruct((n2d, 128), jnp.float32),
        ],
    )(x16, deg_col, theta2d, r2d, w1p, b1.reshape(1, 288),
      w2.astype(jnp.bfloat16), b2.reshape(1, 1024), wf.astype(jnp.bfloat16),
      bf.reshape(1, 64))


def _tc_edge(fs, cs, pxs, pys, fd, pxd, pyd, n2d):
    nb = 25
    rb = n2d // nb

    def body(fs_ref, cs_ref, pxs_ref, pys_ref, fd_ref, pxd_ref, pyd_ref,
             a1_ref, a2_ref):
        df = jnp.abs(fs_ref[...] - fd_ref[...])
        a1_ref[...] = jnp.where(df == 1.0,
                                1.0 / jnp.maximum(cs_ref[...], 1.0), 0.0)
        d2 = (pxd_ref[...] - pxs_ref[...]) ** 2 + \
             (pyd_ref[...] - pys_ref[...]) ** 2
        disp = jnp.sqrt(jnp.maximum(d2, 1e-12))
        vel = disp / jnp.where(df == 2.0, df, 1.0)
        a2_ref[...] = jnp.where(df == 2.0, jnp.exp(-vel / 8.5), 0.0)

    spec = pl.BlockSpec((rb, 128), lambda i: (i, 0))
    return pl.pallas_call(
        body,
        grid=(nb,),
        in_specs=[spec] * 7,
        out_specs=[spec] * 2,
        out_shape=[jax.ShapeDtypeStruct((n2d, 128), jnp.float32)] * 2,
    )(fs, cs, pxs, pys, fd, pxd, pyd)


def _tc_mid(agg1a, agg1b, ya, yb, dinv_col, isrc, idst, gcn1_b, gcn2_w, n):
    nb = n // _R

    def body(aggA_ref, aggB_ref, ya_ref, yb_ref, dinv_ref, is_ref, id_ref,
             b1_ref, w2_ref, y2a_ref, y2b_ref, a2_ref):
        dinv = dinv_ref[...]
        agg = jnp.concatenate([aggA_ref[...], aggB_ref[...]], axis=1)
        y1 = jnp.concatenate([ya_ref[...], yb_ref[...]], axis=1)
        g1 = dinv * (agg + y1) + b1_ref[...]
        fs = is_ref[:, 0:1]
        cs = is_ref[:, 1:2]
        pxs = is_ref[:, 2:3]
        pys = is_ref[:, 3:4]
        fd = id_ref[:, 0:1]
        pxd = id_ref[:, 2:3]
        pyd = id_ref[:, 3:4]
        df = jnp.abs(fs - fd)
        a1 = jnp.where(df == 1.0, 1.0 / jnp.maximum(cs, 1.0), 0.0)
        x1 = jnp.maximum(g1 * a1, 0.0)
        d2 = (pxd - pxs) ** 2 + (pyd - pys) ** 2
        disp = jnp.sqrt(jnp.maximum(d2, 1e-12))
        vel = disp / jnp.where(df == 2.0, df, 1.0)
        a2_ref[...] = jnp.where(df == 2.0, jnp.exp(-vel / 8.5), 0.0)
        y2 = dinv * jnp.dot(x1, w2_ref[...], precision=_HI)
        y2a_ref[...] = y2[:, :32]
        y2b_ref[...] = y2[:, 32:]

    return pl.pallas_call(
        body,
        grid=(nb,),
        in_specs=[
            pl.BlockSpec((_R, 32), lambda i: (i, 0)),
            pl.BlockSpec((_R, 32), lambda i: (i, 0)),
            pl.BlockSpec((_R, 32), lambda i: (i, 0)),
            pl.BlockSpec((_R, 32), lambda i: (i, 0)),
            pl.BlockSpec((_R, 1), lambda i: (i, 0)),
            pl.BlockSpec((_R, 16), lambda i: (i, 0)),
            pl.BlockSpec((_R, 16), lambda i: (i, 0)),
            pl.BlockSpec((1, 64), lambda i: (0, 0)),
            pl.BlockSpec((64, 64), lambda i: (0, 0)),
        ],
        out_specs=[
            pl.BlockSpec((_R, 32), lambda i: (i, 0)),
            pl.BlockSpec((_R, 32), lambda i: (i, 0)),
            pl.BlockSpec((_R, 1), lambda i: (i, 0)),
        ],
        out_shape=[
            jax.ShapeDtypeStruct((n, 32), jnp.float32),
            jax.ShapeDtypeStruct((n, 32), jnp.float32),
            jax.ShapeDtypeStruct((n, 1), jnp.float32),
        ],
    )(agg1a, agg1b, ya, yb, dinv_col, isrc, idst,
      gcn1_b.reshape(1, 64), gcn2_w)


def _tc_head(agg2a, agg2b, y2a, y2b, dinv_col, a2_col, gcn2_b, fc_w, fc_b,
             out_w, out_b, n):
    nb = n // _R

    def body(aggA_ref, aggB_ref, ya_ref, yb_ref, dinv_ref, a2_ref, b2_ref,
             fw_ref, fb_ref, ow_ref, ob_ref, o_ref):
        dinv = dinv_ref[...]
        agg = jnp.concatenate([aggA_ref[...], aggB_ref[...]], axis=1)
        y2 = jnp.concatenate([ya_ref[...], yb_ref[...]], axis=1)
        g2 = dinv * (agg + y2) + b2_ref[...]
        x2 = jnp.maximum(g2 * a2_ref[...], 0.0)
        h = jnp.maximum(jnp.dot(x2, fw_ref[...], precision=_HI) + fb_ref[...],
                        0.0)
        logit = jnp.dot(h, ow_ref[...], precision=_HI) + ob_ref[...]
        o_ref[...] = 1.0 / (1.0 + jnp.exp(-logit))

    return pl.pallas_call(
        body,
        grid=(nb,),
        in_specs=[
            pl.BlockSpec((_R, 32), lambda i: (i, 0)),
            pl.BlockSpec((_R, 32), lambda i: (i, 0)),
            pl.BlockSpec((_R, 32), lambda i: (i, 0)),
            pl.BlockSpec((_R, 32), lambda i: (i, 0)),
            pl.BlockSpec((_R, 1), lambda i: (i, 0)),
            pl.BlockSpec((_R, 1), lambda i: (i, 0)),
            pl.BlockSpec((1, 64), lambda i: (0, 0)),
            pl.BlockSpec((64, 32), lambda i: (0, 0)),
            pl.BlockSpec((1, 32), lambda i: (0, 0)),
            pl.BlockSpec((32, 1), lambda i: (0, 0)),
            pl.BlockSpec((1, 1), lambda i: (0, 0)),
        ],
        out_specs=pl.BlockSpec((_R, 1), lambda i: (i, 0)),
        out_shape=jax.ShapeDtypeStruct((n, 1), jnp.float32),
    )(agg2a, agg2b, y2a, y2b, dinv_col, a2_col, gcn2_b.reshape(1, 64), fc_w,
      fc_b.reshape(1, 32), out_w, out_b.reshape(1, 1))


def kernel(x, edge_index, aux_features, conv1_w, conv1_b, conv2_w, conv2_b,
           fcfe_w, fcfe_b, gcn1_w, gcn1_b, gcn2_w, gcn2_b,
           fc_w, fc_b, out_w, out_b):
    n = x.shape[0]
    e = edge_index.shape[1]
    e_pad = -(-e // (_NS * _K)) * (_NS * _K)

    x16 = jnp.zeros((n, 16), jnp.float32).at[:, :9].set(x.reshape(n, 9))
    w1p, b1, w2, b2, wf, bf = _fold_weights(
        conv1_w, conv1_b, conv2_w, conv2_b, fcfe_w, fcfe_b, gcn1_w)
    src = edge_index[0]
    dst = edge_index[1]
    pad = e_pad - e
    pad_g = jnp.zeros((pad,), jnp.int32)
    pad_s = jnp.full((pad,), n, jnp.int32)
    src_g = jnp.concatenate([src, pad_g])
    dst_g = jnp.concatenate([dst, pad_g])
    src_s = jnp.concatenate([src, pad_s])
    dst_s = jnp.concatenate([dst, pad_s])
    scat_idx = jnp.concatenate([dst_s, src_s])
    ones_rows = jnp.zeros((_K, 16), jnp.float32).at[:, 0].set(1.0)
    zpt = _acc_geom(n)[2]
    zrows16 = jnp.zeros((zpt, 16), jnp.float32)
    zrows32 = jnp.zeros((zpt, 32), jnp.float32)

    n2d = e_pad // 128
    npad = e_pad - n

    def to2d(col, fill=0.0):
        return jnp.concatenate(
            [col, jnp.full((npad,), fill, jnp.float32)]).reshape(n2d, 128)

    theta2d = to2d(aux_features[:, 0])
    r2d = to2d(aux_features[:, 1])

    deg16, cnt16 = _sc_s1(scat_idx, ones_rows, zrows16, n, e_pad)
    deg_col = deg16[:n, 0:1]
    ya, yb, dinv_col, px2d, py2d = _tc_main(
        x16, deg_col, theta2d, r2d, w1p, b1, w2, b2, wf, bf, n, n2d)
    px_col = px2d.reshape(e_pad, 1)
    py_col = py2d.reshape(e_pad, 1)
    nodeinfo = jnp.concatenate(
        [aux_features[:, 2:3], cnt16[:n, 0:1], px_col[:n], py_col[:n],
         jnp.zeros((n, 12), jnp.float32)], axis=1)
    agg1a, agg1b, isrc, idst = _sc_agg(ya, yb, src_g, dst_g, dst_s,
                                       nodeinfo, zrows32, n, e_pad,
                                       with_info=True)
    y2a, y2b, a2_col = _tc_mid(agg1a, agg1b, ya, yb, dinv_col, isrc, idst,
                               gcn1_b, gcn2_w, n)
    agg2a, agg2b = _sc_agg(y2a, y2b, src_g, dst_g, dst_s, nodeinfo,
                           zrows32, n, e_pad, with_info=False)
    out = _tc_head(agg2a, agg2b, y2a, y2b, dinv_col, a2_col, gcn2_b,
                   fc_w, fc_b, out_w, out_b, n)
    return out
